# baseline (speedup 1.0000x reference)
import math

import jax
import jax.numpy as jnp
from jax import lax
from jax.experimental import pallas as pl
from jax.experimental.pallas import tpu as pltpu
from jax.experimental.pallas import tpu_sc as plsc

VOCAB = 1000000
EMB = 64
B = 4096
L = 200
SCALE = math.sqrt(EMB)

NC, NS = 2, 16
NW = NC * NS
BB = B // NW
SLAB = EMB * BB
TROW = SLAB // 8


def _emb_kernel(tok_hbm, table_hbm, out_hbm, idx_v, lin0, lin1, g0, g1,
                t0, t1, gsem0, gsem1, osem0, osem1):
    wid = lax.axis_index("s") * NC + lax.axis_index("c")
    lin_p = (lin0, lin1)
    g_p = (g0, g1)
    t_p = (t0, t1)
    gsem_p = (gsem0, gsem1)
    osem_p = (osem0, osem1)

    iota = lax.iota(jnp.int32, 16)
    row16 = (iota * L).astype(jnp.int32)
    sc128 = (iota * BB).astype(jnp.int32)

    pltpu.sync_copy(tok_hbm.at[pl.ds(wid * BB * L, BB * L)], idx_v)

    def repack(c, p):
        def rep_body(j, _):
            v = plsc.load_gather(idx_v, [row16 + (j * 16 * L + c)])
            lin_p[p][pl.ds(j * 16, 16)] = v
            return 0

        lax.fori_loop(0, BB // 16, rep_body, 0, unroll=8)

    def fire_gather(p):
        pltpu.async_copy(table_hbm.at[lin_p[p]], g_p[p], gsem_p[p])

    def drain_gather(p):
        pltpu.make_async_copy(table_hbm.at[lin_p[p]], g_p[p],
                              gsem_p[p]).wait()

    def fire_out(c, p):
        for t in range(8):
            pltpu.async_copy(t_p[p].at[pl.ds(t * TROW, TROW)],
                             out_hbm.at[c, t, wid], osem_p[p])

    def drain_out(p):
        for t in range(8):
            pltpu.make_async_copy(t_p[p].at[pl.ds(t * TROW, TROW)],
                                  out_hbm.at[0, t, 0], osem_p[p]).wait()

    repack(0, 0)
    fire_gather(0)
    repack(1, 1)
    fire_gather(1)

    def pair_body(i, _):
        for p in range(2):
            c = 2 * i + p
            drain_gather(p)

            @pl.when(i >= 1)
            def _():
                drain_out(p)

            def tr_body(r, _):
                for cc in range(EMB // 16):
                    x = g_p[p][r, pl.ds(cc * 16, 16)] * SCALE
                    plsc.store_scatter(t_p[p],
                                       [sc128 + (cc * 16 * BB + r)], x)
                return 0

            lax.fori_loop(0, BB, tr_body, 0, unroll=2)
            fire_out(c, p)

            @pl.when(c + 2 < L)
            def _():
                repack(c + 2, p)
                fire_gather(p)

        return 0

    lax.fori_loop(0, L // 2, pair_body, 0)
    drain_out(0)
    drain_out(1)


@jax.jit
def kernel(tokens, table):
    mesh = plsc.VectorSubcoreMesh(core_axis_name="c", subcore_axis_name="s")
    out4 = pl.kernel(
        _emb_kernel,
        out_type=jax.ShapeDtypeStruct((L, 8, NW, TROW), jnp.float32),
        mesh=mesh,
        scratch_types=[
            pltpu.VMEM((BB * L,), jnp.int32),
            pltpu.VMEM((BB,), jnp.int32),
            pltpu.VMEM((BB,), jnp.int32),
            pltpu.VMEM((BB, EMB), jnp.float32),
            pltpu.VMEM((BB, EMB), jnp.float32),
            pltpu.VMEM((SLAB,), jnp.float32),
            pltpu.VMEM((SLAB,), jnp.float32),
            pltpu.SemaphoreType.DMA,
            pltpu.SemaphoreType.DMA,
            pltpu.SemaphoreType.DMA,
            pltpu.SemaphoreType.DMA,
        ],
        compiler_params=pltpu.CompilerParams(use_tc_tiling_on_sc=False,
                                             needs_layout_passes=False),
    )(tokens.reshape(B * L).astype(jnp.int32), table)
    r5 = out4.reshape(L, 8, NW, 8, BB)
    return r5.transpose(2, 4, 0, 1, 3).reshape(B, L, EMB)


# speedup vs baseline: 1.5107x; 1.5107x over previous
import math

import jax
import jax.numpy as jnp
from jax import lax
from jax.experimental import pallas as pl
from jax.experimental.pallas import tpu as pltpu
from jax.experimental.pallas import tpu_sc as plsc

VOCAB = 1000000
EMB = 64
B = 4096
L = 200
SCALE = math.sqrt(EMB)

NC, NS = 2, 16
NW = NC * NS
BB = B // NW
SLAB = EMB * BB
TROW = SLAB // 8


def _emb_kernel(tok_hbm, table_hbm, out_hbm, idx_v, lin0, lin1, g0, g1,
                t0, t1, gsem0, gsem1, osem0, osem1):
    wid = lax.axis_index("s") * NC + lax.axis_index("c")
    lin_p = (lin0, lin1)
    g_p = (g0, g1)
    t_p = (t0, t1)
    gsem_p = (gsem0, gsem1)
    osem_p = (osem0, osem1)

    iota = lax.iota(jnp.int32, 16)
    row16 = (iota * L).astype(jnp.int32)
    evec = [iota + 16 * cc for cc in range(EMB // 16)]
    e128 = [(iota + 16 * cc) * BB for cc in range(EMB // 16)]

    pltpu.sync_copy(tok_hbm.at[pl.ds(wid * BB * L, BB * L)], idx_v)

    def repack(c, p):
        def rep_body(j, _):
            v = plsc.load_gather(idx_v, [row16 + (j * 16 * L + c)])
            lin_p[p][pl.ds(j * 16, 16)] = v
            return 0

        lax.fori_loop(0, BB // 16, rep_body, 0, unroll=8)

    def fire_gather(p):
        pltpu.async_copy(table_hbm.at[lin_p[p]], g_p[p], gsem_p[p])

    def drain_gather(p):
        pltpu.make_async_copy(table_hbm.at[lin_p[p]], g_p[p],
                              gsem_p[p]).wait()

    def fire_out(c, p):
        for t in range(8):
            pltpu.async_copy(t_p[p].at[pl.ds(t * TROW, TROW)],
                             out_hbm.at[c, t, wid], osem_p[p])

    def drain_out(p):
        for t in range(8):
            pltpu.make_async_copy(t_p[p].at[pl.ds(t * TROW, TROW)],
                                  out_hbm.at[0, t, 0], osem_p[p]).wait()

    repack(0, 0)
    fire_gather(0)
    repack(1, 1)
    fire_gather(1)

    def pair_body(i, _):
        for p in range(2):
            c = 2 * i + p
            drain_gather(p)

            @pl.when(i >= 1)
            def _():
                drain_out(p)

            def tr_body(k, mvec):
                for cc in range(EMB // 16):
                    x = plsc.load_gather(g_p[p], [mvec, evec[cc]]) * SCALE
                    plsc.store_scatter(t_p[p], [e128[cc] + mvec], x)
                return (mvec + 1) & 127

            lax.fori_loop(0, BB, tr_body, iota, unroll=4)
            fire_out(c, p)

            @pl.when(c + 2 < L)
            def _():
                repack(c + 2, p)
                fire_gather(p)

        return 0

    lax.fori_loop(0, L // 2, pair_body, 0)
    drain_out(0)
    drain_out(1)


@jax.jit
def kernel(tokens, table):
    mesh = plsc.VectorSubcoreMesh(core_axis_name="c", subcore_axis_name="s")
    out4 = pl.kernel(
        _emb_kernel,
        out_type=jax.ShapeDtypeStruct((L, 8, NW, TROW), jnp.float32),
        mesh=mesh,
        scratch_types=[
            pltpu.VMEM((BB * L,), jnp.int32),
            pltpu.VMEM((BB,), jnp.int32),
            pltpu.VMEM((BB,), jnp.int32),
            pltpu.VMEM((BB, EMB), jnp.float32),
            pltpu.VMEM((BB, EMB), jnp.float32),
            pltpu.VMEM((SLAB,), jnp.float32),
            pltpu.VMEM((SLAB,), jnp.float32),
            pltpu.SemaphoreType.DMA,
            pltpu.SemaphoreType.DMA,
            pltpu.SemaphoreType.DMA,
            pltpu.SemaphoreType.DMA,
        ],
        compiler_params=pltpu.CompilerParams(use_tc_tiling_on_sc=False,
                                             needs_layout_passes=False),
    )(tokens.reshape(B * L).astype(jnp.int32), table)
    r5 = out4.reshape(L, 8, NW, 8, BB)
    return r5.transpose(2, 4, 0, 1, 3).reshape(B, L, EMB)


# speedup vs baseline: 1.5252x; 1.0096x over previous
import math

import jax
import jax.numpy as jnp
from jax import lax
from jax.experimental import pallas as pl
from jax.experimental.pallas import tpu as pltpu
from jax.experimental.pallas import tpu_sc as plsc

VOCAB = 1000000
EMB = 64
B = 4096
L = 200
SCALE = math.sqrt(EMB)

NC, NS = 2, 16
NW = NC * NS
BB = B // NW
SLAB = EMB * BB
TROW = SLAB // 8


def _emb_kernel(tok_hbm, table_hbm, out_hbm, idx_v, lin0, lin1, g0, g1,
                t0, t1, gsem0, gsem1, osem0, osem1):
    wid = lax.axis_index("s") * NC + lax.axis_index("c")
    lin_p = (lin0, lin1)
    g_p = (g0, g1)
    t_p = (t0, t1)
    gsem_p = (gsem0, gsem1)
    osem_p = (osem0, osem1)

    iota = lax.iota(jnp.int32, 16)
    row16 = (iota * L).astype(jnp.int32)
    evec = [iota + 16 * cc for cc in range(EMB // 16)]
    trow = [(iota // 8) + 2 * cc for cc in range(EMB // 16)]
    tcol = (iota % 8) * BB

    pltpu.sync_copy(tok_hbm.at[pl.ds(wid * BB * L, BB * L)], idx_v)

    def repack(c, p):
        def rep_body(j, _):
            v = plsc.load_gather(idx_v, [row16 + (j * 16 * L + c)])
            lin_p[p][pl.ds(j * 16, 16)] = v
            return 0

        lax.fori_loop(0, BB // 16, rep_body, 0, unroll=8)

    def fire_gather(p):
        pltpu.async_copy(table_hbm.at[lin_p[p]], g_p[p], gsem_p[p])

    def drain_gather(p):
        pltpu.make_async_copy(table_hbm.at[lin_p[p]], g_p[p],
                              gsem_p[p]).wait()

    def fire_out(c, p):
        pltpu.async_copy(t_p[p], out_hbm.at[c, :, wid], osem_p[p])

    def drain_out(p):
        pltpu.make_async_copy(t_p[p], out_hbm.at[0, :, 0],
                              osem_p[p]).wait()

    repack(0, 0)
    fire_gather(0)
    repack(1, 1)
    fire_gather(1)

    def pair_body(i, _):
        for p in range(2):
            c = 2 * i + p
            drain_gather(p)

            @pl.when(i >= 1)
            def _():
                drain_out(p)

            def tr_body(k, mvec):
                for cc in range(EMB // 16):
                    x = plsc.load_gather(g_p[p], [mvec, evec[cc]]) * SCALE
                    plsc.store_scatter(t_p[p], [trow[cc], tcol + mvec], x)
                return (mvec + 1) & 127

            lax.fori_loop(0, BB, tr_body, iota, unroll=8)
            fire_out(c, p)

            @pl.when(c + 2 < L)
            def _():
                repack(c + 2, p)
                fire_gather(p)

        return 0

    lax.fori_loop(0, L // 2, pair_body, 0)
    drain_out(0)
    drain_out(1)


@jax.jit
def kernel(tokens, table):
    mesh = plsc.VectorSubcoreMesh(core_axis_name="c", subcore_axis_name="s")
    out4 = pl.kernel(
        _emb_kernel,
        out_type=jax.ShapeDtypeStruct((L, 8, NW, TROW), jnp.float32),
        mesh=mesh,
        scratch_types=[
            pltpu.VMEM((BB * L,), jnp.int32),
            pltpu.VMEM((BB,), jnp.int32),
            pltpu.VMEM((BB,), jnp.int32),
            pltpu.VMEM((BB, EMB), jnp.float32),
            pltpu.VMEM((BB, EMB), jnp.float32),
            pltpu.VMEM((8, TROW), jnp.float32),
            pltpu.VMEM((8, TROW), jnp.float32),
            pltpu.SemaphoreType.DMA,
            pltpu.SemaphoreType.DMA,
            pltpu.SemaphoreType.DMA,
            pltpu.SemaphoreType.DMA,
        ],
        compiler_params=pltpu.CompilerParams(use_tc_tiling_on_sc=False,
                                             needs_layout_passes=False),
    )(tokens.reshape(B * L).astype(jnp.int32), table)
    r5 = out4.reshape(L, 8, NW, 8, BB)
    return r5.transpose(2, 4, 0, 1, 3).reshape(B, L, EMB)


# speedup vs baseline: 2.1365x; 1.4008x over previous
import math

import jax
import jax.numpy as jnp
from jax import lax
from jax.experimental import pallas as pl
from jax.experimental.pallas import tpu as pltpu
from jax.experimental.pallas import tpu_sc as plsc

VOCAB = 1000000
EMB = 64
B = 4096
L = 200
SCALE = math.sqrt(EMB)

NC, NS = 2, 16
NW = NC * NS
BB = B // NW
SLAB = EMB * BB
TROW = SLAB // 8


def _emb_kernel(tok_hbm, table_hbm, out_hbm, idx_v, lin0, lin1, g0, g1,
                t0, t1, gsem0, gsem1, osem0, osem1):
    wid = lax.axis_index("s") * NC + lax.axis_index("c")
    lin_p = (lin0, lin1)
    g_p = (g0, g1)
    t_p = (t0, t1)
    gsem_p = (gsem0, gsem1)
    osem_p = (osem0, osem1)

    iota = lax.iota(jnp.int32, 16)
    row16 = (iota * L).astype(jnp.int32)
    evec = [iota + 16 * cc for cc in range(EMB // 16)]
    trow = [(iota // 8) + 2 * cc for cc in range(EMB // 16)]
    tcol = (iota % 8) * BB

    pltpu.sync_copy(tok_hbm.at[pl.ds(wid * BB * L, BB * L)], idx_v)

    def repack(c, p):
        @plsc.parallel_loop(0, BB // 16, unroll=8)
        def _(j):
            v = plsc.load_gather(idx_v, [row16 + (j * 16 * L + c)])
            lin_p[p][pl.ds(j * 16, 16)] = v

    def fire_gather(p):
        pltpu.async_copy(table_hbm.at[lin_p[p]], g_p[p], gsem_p[p])

    def drain_gather(p):
        pltpu.make_async_copy(table_hbm.at[lin_p[p]], g_p[p],
                              gsem_p[p]).wait()

    def fire_out(c, p):
        pltpu.async_copy(t_p[p], out_hbm.at[c, :, wid], osem_p[p])

    def drain_out(p):
        pltpu.make_async_copy(t_p[p], out_hbm.at[0, :, 0],
                              osem_p[p]).wait()

    repack(0, 0)
    fire_gather(0)
    repack(1, 1)
    fire_gather(1)

    def pair_body(i, _):
        for p in range(2):
            c = 2 * i + p
            drain_gather(p)

            @pl.when(i >= 1)
            def _():
                drain_out(p)

            @plsc.parallel_loop(0, BB, unroll=8, carry=iota)
            def _(k, mvec):
                for cc in range(EMB // 16):
                    x = plsc.load_gather(g_p[p], [mvec, evec[cc]]) * SCALE
                    plsc.store_scatter(t_p[p], [trow[cc], tcol + mvec], x)
                return (mvec + 1) & 127
            fire_out(c, p)

            @pl.when(c + 2 < L)
            def _():
                repack(c + 2, p)
                fire_gather(p)

        return 0

    lax.fori_loop(0, L // 2, pair_body, 0)
    drain_out(0)
    drain_out(1)


@jax.jit
def kernel(tokens, table):
    mesh = plsc.VectorSubcoreMesh(core_axis_name="c", subcore_axis_name="s")
    out4 = pl.kernel(
        _emb_kernel,
        out_type=jax.ShapeDtypeStruct((L, 8, NW, TROW), jnp.float32),
        mesh=mesh,
        scratch_types=[
            pltpu.VMEM((BB * L,), jnp.int32),
            pltpu.VMEM((BB,), jnp.int32),
            pltpu.VMEM((BB,), jnp.int32),
            pltpu.VMEM((BB, EMB), jnp.float32),
            pltpu.VMEM((BB, EMB), jnp.float32),
            pltpu.VMEM((8, TROW), jnp.float32),
            pltpu.VMEM((8, TROW), jnp.float32),
            pltpu.SemaphoreType.DMA,
            pltpu.SemaphoreType.DMA,
            pltpu.SemaphoreType.DMA,
            pltpu.SemaphoreType.DMA,
        ],
        compiler_params=pltpu.CompilerParams(use_tc_tiling_on_sc=False,
                                             needs_layout_passes=False),
    )(tokens.reshape(B * L).astype(jnp.int32), table)
    r5 = out4.reshape(L, 8, NW, 8, BB)
    return r5.transpose(2, 4, 0, 1, 3).reshape(B, L, EMB)
